# Initial kernel scaffold; baseline (speedup 1.0000x reference)
import jax
import jax.numpy as jnp
from jax.experimental import pallas as pl


def kernel(x, phi_a, phi_b, weight, bias):
    raise NotImplementedError("write your pallas kernel here")


# speedup vs baseline: 2.4508x; 2.4508x over previous
import jax
import jax.numpy as jnp
from jax.experimental import pallas as pl
from jax.experimental.pallas import tpu as pltpu

_N = 256
_M = 51529
_CK = 4096
_NC = 13
_MPAD = _CK * _NC
_KOUT = 65536
_W_CHUNK = 16384


def _glcm_body(a_ref, b_ref, pa_ref, pb_ref, out_ref):
    reps = _CK // 128
    pa = jnp.concatenate([pa_ref[...]] * reps, axis=1)
    pb = jnp.concatenate([pb_ref[...]] * reps, axis=1)
    acc = jnp.zeros((_N, _N), jnp.float32)
    for c in range(_NC):
        a_row = a_ref[0, :, c * _CK:(c + 1) * _CK]
        b_row = b_ref[0, :, c * _CK:(c + 1) * _CK]
        sa = (jnp.broadcast_to(a_row, (_N, _CK)) - pa).astype(jnp.bfloat16)
        sb = (jnp.broadcast_to(b_row, (_N, _CK)) - pb).astype(jnp.bfloat16)
        sa = jnp.clip(sa, 0.0, 1.0)
        sb = jnp.clip(sb, 0.0, 1.0)
        acc = acc + jax.lax.dot_general(
            sa, sb, (((1,), (1,)), ((), ())),
            preferred_element_type=jnp.float32)
    out_ref[0] = acc


def _linear_body(g_ref, w_ref, bias_ref, out_ref):
    c = pl.program_id(0)
    g = g_ref[...]
    w = w_ref[...]
    h = _W_CHUNK // 2
    p = jax.lax.dot_general(g[:, :h], w[:h, :], (((1,), (0,)), ((), ())),
                            preferred_element_type=jnp.float32)
    p = p + jax.lax.dot_general(g[:, h:], w[h:, :], (((1,), (0,)), ((), ())),
                                preferred_element_type=jnp.float32)

    @pl.when(c == 0)
    def _():
        out_ref[...] = p

    @pl.when(c > 0)
    def _():
        out_ref[...] = out_ref[...] + p

    @pl.when(c == (_KOUT // _W_CHUNK) - 1)
    def _():
        out_ref[...] = jnp.maximum(out_ref[...] + bias_ref[...], 0.0)


def kernel(x, phi_a, phi_b, weight, bias):
    b_sz = x.shape[0]
    a = x.reshape(b_sz, -1)
    bdiff = a - jnp.pad(a[:, 1:], ((0, 0), (0, 1)))
    pad = _MPAD - _M
    a_p = jnp.pad(a, ((0, 0), (0, pad)), constant_values=-1e9).reshape(
        b_sz, 1, _MPAD)
    b_p = jnp.pad(bdiff, ((0, 0), (0, pad)), constant_values=-1e9).reshape(
        b_sz, 1, _MPAD)
    pa128 = jnp.broadcast_to(phi_a[:, None], (_N, 128))
    pb128 = jnp.broadcast_to(phi_b[:, None], (_N, 128))

    glcm = pl.pallas_call(
        _glcm_body,
        grid=(b_sz,),
        in_specs=[
            pl.BlockSpec((1, 1, _MPAD), lambda b: (b, 0, 0)),
            pl.BlockSpec((1, 1, _MPAD), lambda b: (b, 0, 0)),
            pl.BlockSpec((_N, 128), lambda b: (0, 0)),
            pl.BlockSpec((_N, 128), lambda b: (0, 0)),
        ],
        out_specs=pl.BlockSpec((1, _N, _N), lambda b: (b, 0, 0)),
        out_shape=jax.ShapeDtypeStruct((b_sz, _N, _N), jnp.float32),
        compiler_params=pltpu.CompilerParams(
            dimension_semantics=(pltpu.PARALLEL,),
        ),
    )(a_p, b_p, pa128, pb128)

    g = glcm.reshape(b_sz, _KOUT)
    nsteps = _KOUT // _W_CHUNK
    out = pl.pallas_call(
        _linear_body,
        grid=(nsteps,),
        in_specs=[
            pl.BlockSpec((b_sz, _W_CHUNK), lambda c: (0, c)),
            pl.BlockSpec((_W_CHUNK, 32), lambda c: (c, 0)),
            pl.BlockSpec((1, 32), lambda c: (0, 0)),
        ],
        out_specs=pl.BlockSpec((b_sz, 32), lambda c: (0, 0)),
        out_shape=jax.ShapeDtypeStruct((b_sz, 32), jnp.float32),
        compiler_params=pltpu.CompilerParams(
            dimension_semantics=(pltpu.ARBITRARY,),
        ),
    )(g, weight, bias.reshape(1, 32))
    return out
